# Initial kernel scaffold; baseline (speedup 1.0000x reference)
import jax
import jax.numpy as jnp
from jax.experimental import pallas as pl


def kernel(x, W):
    raise NotImplementedError("write your pallas kernel here")


# speedup vs baseline: 1.0060x; 1.0060x over previous
import jax
import jax.numpy as jnp
from jax.experimental import pallas as pl


def kernel(x, W):
    B = x.shape[0]
    S = x.shape[-2]
    D = x.shape[-1]

    W2 = W[: S * D].reshape(S, D)

    SBLK = 256
    n_blocks = S // SBLK

    def body(w_ref, o_ref):
        o_ref[...] = jnp.broadcast_to(w_ref[...][None, None], (B, 1, SBLK, D))

    out = pl.pallas_call(
        body,
        grid=(n_blocks,),
        in_specs=[pl.BlockSpec((SBLK, D), lambda i: (i, 0))],
        out_specs=pl.BlockSpec((B, 1, SBLK, D), lambda i: (0, 0, i, 0)),
        out_shape=jax.ShapeDtypeStruct((B, 1, S, D), W.dtype),
    )(W2)
    return out


# speedup vs baseline: 1.0212x; 1.0151x over previous
import jax
import jax.numpy as jnp
from jax.experimental import pallas as pl


def kernel(x, W):
    B = x.shape[0]
    S = x.shape[-2]
    D = x.shape[-1]

    W2 = W[: S * D].reshape(S, D)

    SBLK = 512
    n_blocks = S // SBLK

    def body(w_ref, o_ref):
        o_ref[...] = jnp.broadcast_to(w_ref[...][None, None], (B, 1, SBLK, D))

    out = pl.pallas_call(
        body,
        grid=(n_blocks,),
        in_specs=[pl.BlockSpec((SBLK, D), lambda i: (i, 0))],
        out_specs=pl.BlockSpec((B, 1, SBLK, D), lambda i: (0, 0, i, 0)),
        out_shape=jax.ShapeDtypeStruct((B, 1, S, D), W.dtype),
    )(W2)
    return out
